# baseline (speedup 1.0000x reference)
import jax
import jax.numpy as jnp
from jax import lax
from jax.experimental import pallas as pl

_NUM_TAGS = 64
_HIGH = 5.0
_LOW = -5.0

_B, _S, _K = 256, 40, 30
_GB = 16
_NSTEP = _B // _GB


def _score_expand_body(labels_ref, out_ref):
    tags = lax.broadcasted_iota(jnp.int32, (_GB, _S, _K, _NUM_TAGS), 3)
    out_ref[...] = jnp.where(
        tags == labels_ref[...][:, :, :, None], _HIGH, _LOW
    )


_score_expand = pl.pallas_call(
    _score_expand_body,
    out_shape=jax.ShapeDtypeStruct((_B, _S, _K, _NUM_TAGS), jnp.float32),
    grid=(_NSTEP,),
    in_specs=[pl.BlockSpec((_GB, _S, _K), lambda i: (i, 0, 0))],
    out_specs=pl.BlockSpec((_GB, _S, _K, _NUM_TAGS), lambda i: (i, 0, 0, 0)),
)


def kernel(arg_labels):
    return _score_expand(arg_labels.astype(jnp.int32))
